# Initial kernel scaffold; baseline (speedup 1.0000x reference)
import jax
import jax.numpy as jnp
from jax.experimental import pallas as pl


def kernel(atom_in_fea, nbr_fea, nbr_fea_idx, W_fc, b_fc, bn1_gamma, bn1_beta, bn2_gamma, bn2_beta):
    raise NotImplementedError("write your pallas kernel here")


# speedup vs baseline: 1.0484x; 1.0484x over previous
import functools

import jax
import jax.numpy as jnp
from jax import lax
from jax.experimental import pallas as pl
from jax.experimental.pallas import tpu as pltpu
from jax.experimental.pallas import tpu_sc as plsc

N_ATOMS = 50000
M_NBRS = 16
F = 64
EF = 41
OUT2 = 128
EPS = 1e-5

R = N_ATOMS * M_NBRS
BN = 400
BR = BN * M_NBRS
NBLK = N_ATOMS // BN

NW = 32
CHUNK = 128
CHUNKS_PER_W = 200
R_PAD = NW * CHUNKS_PER_W * CHUNK


def _sc_gather(table, idx2d):
    mesh = plsc.VectorSubcoreMesh(core_axis_name="c", subcore_axis_name="s")

    @functools.partial(
        pl.kernel,
        mesh=mesh,
        out_type=jax.ShapeDtypeStruct((R_PAD, OUT2), jnp.float32),
        scratch_types=[
            pltpu.VMEM((CHUNKS_PER_W, CHUNK), jnp.int32),
            pltpu.VMEM((CHUNK, OUT2), jnp.float32),
            pltpu.SemaphoreType.DMA,
        ],
    )
    def k(table_hbm, idx_hbm, out_hbm, idx_v, rows_v, sem):
        wid = lax.axis_index("s") * 2 + lax.axis_index("c")
        pltpu.sync_copy(idx_hbm.at[pl.ds(wid * CHUNKS_PER_W, CHUNKS_PER_W)], idx_v)
        base = wid * (CHUNKS_PER_W * CHUNK)

        def body(j, carry):
            pltpu.async_copy(table_hbm.at[idx_v.at[j]], rows_v, sem).wait()
            pltpu.sync_copy(rows_v, out_hbm.at[pl.ds(base + j * CHUNK, CHUNK)])
            return carry

        lax.fori_loop(0, CHUNKS_PER_W, body, 0)

    return k(table, idx2d)


def _rep16(t2):
    c = t2.shape[-1]
    return jnp.broadcast_to(t2[:, None, :], (BN, M_NBRS, c)).reshape(BR, c)


def _gated(a_ref, g_ref, e_ref, ws_ref, wn_ref, we_ref):
    t1 = jnp.dot(g_ref[...], wn_ref[...], preferred_element_type=jnp.float32)
    t1 += jnp.dot(e_ref[...], we_ref[...], preferred_element_type=jnp.float32)
    t2 = jnp.dot(a_ref[...], ws_ref[...], preferred_element_type=jnp.float32)
    return t1 + _rep16(t2)


def _softplus(x):
    return jnp.maximum(x, 0.0) + jnp.log1p(jnp.exp(-jnp.abs(x)))


def _body_stats(a_ref, g_ref, e_ref, ws_ref, wn_ref, we_ref, acc_ref):
    i = pl.program_id(0)
    y = _gated(a_ref, g_ref, e_ref, ws_ref, wn_ref, we_ref)
    ps = jnp.sum(y, axis=0, keepdims=True)
    pq = jnp.sum(y * y, axis=0, keepdims=True)
    part = jnp.concatenate([ps, pq], axis=0)

    @pl.when(i == 0)
    def _():
        acc_ref[...] = jnp.zeros_like(acc_ref)

    acc_ref[...] += part


def _body_apply(a_ref, g_ref, e_ref, ws_ref, wn_ref, we_ref, c1_ref,
                s_ref, acc2_ref):
    i = pl.program_id(0)
    y = _gated(a_ref, g_ref, e_ref, ws_ref, wn_ref, we_ref)
    z = y * c1_ref[0:1, :] + c1_ref[1:2, :]
    f = jax.nn.sigmoid(z[:, :F])
    c = _softplus(z[:, F:])
    prod = (f * c).reshape(BN, M_NBRS, F)
    s_blk = jnp.sum(prod, axis=1)
    s_ref[...] = s_blk
    ps = jnp.sum(s_blk, axis=0, keepdims=True)
    pq = jnp.sum(s_blk * s_blk, axis=0, keepdims=True)
    part = jnp.concatenate([ps, pq], axis=0)

    @pl.when(i == 0)
    def _():
        acc2_ref[...] = jnp.zeros_like(acc2_ref)

    acc2_ref[...] += part


def _body_final(a_ref, s_ref, c2_ref, out_ref):
    v = s_ref[...] * c2_ref[0:1, :] + c2_ref[1:2, :]
    out_ref[...] = _softplus(a_ref[...] + v)


def _row_spec(rows, cols):
    return pl.BlockSpec((rows, cols), lambda i: (i, 0))


def _const_spec(shape):
    return pl.BlockSpec(shape, lambda i: tuple(0 for _ in shape))


def _tc_pass1(A, G, E2, WsT, WnT, WeT, interpret=False):
    return pl.pallas_call(
        _body_stats,
        grid=(NBLK,),
        in_specs=[
            _row_spec(BN, F),
            _row_spec(BR, OUT2),
            _row_spec(BR, EF),
            _const_spec((F, OUT2)),
            _const_spec((OUT2, OUT2)),
            _const_spec((EF, OUT2)),
        ],
        out_specs=_const_spec((2, OUT2)),
        out_shape=jax.ShapeDtypeStruct((2, OUT2), jnp.float32),
        interpret=interpret,
    )(A, G, E2, WsT, WnT, WeT)


def _tc_pass2(A, G, E2, WsT, WnT, WeT, coef1, interpret=False):
    return pl.pallas_call(
        _body_apply,
        grid=(NBLK,),
        in_specs=[
            _row_spec(BN, F),
            _row_spec(BR, OUT2),
            _row_spec(BR, EF),
            _const_spec((F, OUT2)),
            _const_spec((OUT2, OUT2)),
            _const_spec((EF, OUT2)),
            _const_spec((2, OUT2)),
        ],
        out_specs=[_row_spec(BN, F), _const_spec((2, F))],
        out_shape=[
            jax.ShapeDtypeStruct((N_ATOMS, F), jnp.float32),
            jax.ShapeDtypeStruct((2, F), jnp.float32),
        ],
        interpret=interpret,
    )(A, G, E2, WsT, WnT, WeT, coef1)


def _tc_pass3(A, s, coef2, interpret=False):
    return pl.pallas_call(
        _body_final,
        grid=(NBLK,),
        in_specs=[
            _row_spec(BN, F),
            _row_spec(BN, F),
            _const_spec((2, F)),
        ],
        out_specs=_row_spec(BN, F),
        out_shape=jax.ShapeDtypeStruct((N_ATOMS, F), jnp.float32),
        interpret=interpret,
    )(A, s, coef2)


def _tc_pipeline(A, G, E2, W_fc, bn1_gamma, bn1_beta, bn2_gamma, bn2_beta,
                 interpret=False):
    WsT = W_fc[:, :F].T
    WnT = jnp.concatenate(
        [W_fc[:, F:2 * F].T, jnp.zeros((OUT2 - F, OUT2), jnp.float32)])
    WeT = W_fc[:, 2 * F:].T
    acc1 = _tc_pass1(A, G, E2, WsT, WnT, WeT, interpret=interpret)
    mean1 = acc1[0] / R
    var1 = acc1[1] / R - mean1 * mean1
    scale1 = bn1_gamma * lax.rsqrt(var1 + EPS)
    shift1 = bn1_beta - mean1 * scale1
    coef1 = jnp.stack([scale1, shift1])
    s, acc2 = _tc_pass2(A, G, E2, WsT, WnT, WeT, coef1, interpret=interpret)
    mean2 = acc2[0] / N_ATOMS
    var2 = acc2[1] / N_ATOMS - mean2 * mean2
    scale2 = bn2_gamma * lax.rsqrt(var2 + EPS)
    shift2 = bn2_beta - mean2 * scale2
    coef2 = jnp.stack([scale2, shift2])
    return _tc_pass3(A, s, coef2, interpret=interpret)


def kernel(atom_in_fea, nbr_fea, nbr_fea_idx, W_fc, b_fc, bn1_gamma,
           bn1_beta, bn2_gamma, bn2_beta):
    del b_fc
    idx = nbr_fea_idx.reshape(-1).astype(jnp.int32)
    idx_pad = jnp.concatenate(
        [idx, jnp.zeros((R_PAD - R,), jnp.int32)]).reshape(R_PAD // CHUNK, CHUNK)
    table = jnp.pad(atom_in_fea, ((0, 0), (0, OUT2 - F)))
    G = _sc_gather(table, idx_pad)
    E2 = nbr_fea.reshape(R, EF)
    return _tc_pipeline(atom_in_fea, G, E2, W_fc, bn1_gamma, bn1_beta,
                        bn2_gamma, bn2_beta)


# speedup vs baseline: 1.1765x; 1.1222x over previous
import functools

import jax
import jax.numpy as jnp
from jax import lax
from jax.experimental import pallas as pl
from jax.experimental.pallas import tpu as pltpu
from jax.experimental.pallas import tpu_sc as plsc

N_ATOMS = 50000
M_NBRS = 16
F = 64
EF = 41
OUT2 = 128
EPS = 1e-5

R = N_ATOMS * M_NBRS
BN = 400
BR = BN * M_NBRS
NBLK = N_ATOMS // BN

NW = 32
CHUNK = 128
CHUNKS_PER_W = 200
ROWS_PER_W = CHUNKS_PER_W * CHUNK
R_PAD = NW * ROWS_PER_W
SUP = 2
SUPROWS = SUP * CHUNK
NSUP = CHUNKS_PER_W // SUP


def _sc_gather(table, idx2d):
    mesh = plsc.VectorSubcoreMesh(core_axis_name="c", subcore_axis_name="s")

    @functools.partial(
        pl.kernel,
        mesh=mesh,
        out_type=jax.ShapeDtypeStruct((R_PAD, OUT2), jnp.float32),
        scratch_types=[
            pltpu.VMEM((CHUNKS_PER_W, CHUNK), jnp.int32),
            pltpu.VMEM((SUPROWS, OUT2), jnp.float32),
            pltpu.VMEM((SUPROWS, OUT2), jnp.float32),
            pltpu.SemaphoreType.DMA,
            pltpu.SemaphoreType.DMA,
            pltpu.SemaphoreType.DMA,
            pltpu.SemaphoreType.DMA,
        ],
    )
    def k(table_hbm, idx_hbm, out_hbm, idx_v, buf0, buf1, g0, g1, w0, w1):
        wid = lax.axis_index("s") * 2 + lax.axis_index("c")
        pltpu.sync_copy(idx_hbm.at[pl.ds(wid * CHUNKS_PER_W, CHUNKS_PER_W)],
                        idx_v)
        base = wid * ROWS_PER_W

        def fire(s, buf, gsem):
            for b in range(SUP):
                pltpu.async_copy(table_hbm.at[idx_v.at[s * SUP + b]],
                                 buf.at[pl.ds(b * CHUNK, CHUNK)], gsem)

        def drain_g(buf, gsem):
            pltpu.make_async_copy(out_hbm.at[pl.ds(base, SUPROWS)], buf,
                                  gsem).wait()

        def wait_w(buf, wsem):
            pltpu.make_async_copy(buf, out_hbm.at[pl.ds(base, SUPROWS)],
                                  wsem).wait()

        def write(s, buf, wsem):
            pltpu.async_copy(buf, out_hbm.at[pl.ds(base + s * SUPROWS,
                                                   SUPROWS)], wsem)

        fire(0, buf0, g0)

        def body(k_, carry):
            s0 = 2 * k_

            @pl.when(k_ > 0)
            def _():
                wait_w(buf1, w1)

            fire(s0 + 1, buf1, g1)
            drain_g(buf0, g0)
            write(s0, buf0, w0)

            @pl.when(k_ < NSUP // 2 - 1)
            def _():
                wait_w(buf0, w0)
                fire(s0 + 2, buf0, g0)

            drain_g(buf1, g1)
            write(s0 + 1, buf1, w1)
            return carry

        lax.fori_loop(0, NSUP // 2, body, 0)
        wait_w(buf0, w0)
        wait_w(buf1, w1)

    return k(table, idx2d)


def _rep16(t2):
    c = t2.shape[-1]
    return jnp.broadcast_to(t2[:, None, :], (BN, M_NBRS, c)).reshape(BR, c)


def _softplus(x):
    return jnp.maximum(x, 0.0) + jnp.log1p(jnp.exp(-jnp.abs(x)))


def _body_proj(a_ref, w_ref, p_ref):
    p_ref[...] = jnp.dot(a_ref[...], w_ref[...],
                         preferred_element_type=jnp.float32)


def _body_stats(a_ref, g_ref, e_ref, ws_ref, we_ref, acc_ref, y_ref):
    i = pl.program_id(0)
    t1 = g_ref[...].astype(jnp.float32)
    t1 += jnp.dot(e_ref[...], we_ref[...], preferred_element_type=jnp.float32)
    t2 = jnp.dot(a_ref[...], ws_ref[...], preferred_element_type=jnp.float32)
    y = t1 + _rep16(t2)
    y_ref[...] = y.astype(jnp.bfloat16)
    ps = jnp.sum(y, axis=0, keepdims=True)
    pq = jnp.sum(y * y, axis=0, keepdims=True)
    part = jnp.concatenate([ps, pq], axis=0)

    @pl.when(i == 0)
    def _():
        acc_ref[...] = jnp.zeros_like(acc_ref)

    acc_ref[...] += part


def _body_apply(y_ref, c1_ref, s_ref, acc2_ref):
    i = pl.program_id(0)
    z = y_ref[...].astype(jnp.float32) * c1_ref[0:1, :] + c1_ref[1:2, :]
    f = jax.nn.sigmoid(z[:, :F])
    c = _softplus(z[:, F:])
    prod = (f * c).reshape(BN, M_NBRS, F)
    s_blk = jnp.sum(prod, axis=1)
    s_ref[...] = s_blk
    ps = jnp.sum(s_blk, axis=0, keepdims=True)
    pq = jnp.sum(s_blk * s_blk, axis=0, keepdims=True)
    part = jnp.concatenate([ps, pq], axis=0)

    @pl.when(i == 0)
    def _():
        acc2_ref[...] = jnp.zeros_like(acc2_ref)

    acc2_ref[...] += part


def _body_final(a_ref, s_ref, c2_ref, out_ref):
    v = s_ref[...] * c2_ref[0:1, :] + c2_ref[1:2, :]
    out_ref[...] = _softplus(a_ref[...] + v)


def _row_spec(rows, cols):
    return pl.BlockSpec((rows, cols), lambda i: (i, 0))


def _const_spec(shape):
    return pl.BlockSpec(shape, lambda i: tuple(0 for _ in shape))


PBLK = 2000


def _tc_project(A, WnT, interpret=False):
    return pl.pallas_call(
        _body_proj,
        grid=(N_ATOMS // PBLK,),
        in_specs=[_row_spec(PBLK, F), _const_spec((F, OUT2))],
        out_specs=_row_spec(PBLK, OUT2),
        out_shape=jax.ShapeDtypeStruct((N_ATOMS, OUT2), jnp.float32),
        interpret=interpret,
    )(A, WnT)


def _tc_pass1(A, G, E2, WsT, WeT, interpret=False):
    return pl.pallas_call(
        _body_stats,
        grid=(NBLK,),
        in_specs=[
            _row_spec(BN, F),
            _row_spec(BR, OUT2),
            _row_spec(BR, EF),
            _const_spec((F, OUT2)),
            _const_spec((EF, OUT2)),
        ],
        out_specs=[_const_spec((2, OUT2)), _row_spec(BR, OUT2)],
        out_shape=[
            jax.ShapeDtypeStruct((2, OUT2), jnp.float32),
            jax.ShapeDtypeStruct((R, OUT2), jnp.bfloat16),
        ],
        interpret=interpret,
    )(A, G, E2, WsT, WeT)


def _tc_pass2(Y, coef1, interpret=False):
    return pl.pallas_call(
        _body_apply,
        grid=(NBLK,),
        in_specs=[
            _row_spec(BR, OUT2),
            _const_spec((2, OUT2)),
        ],
        out_specs=[_row_spec(BN, F), _const_spec((2, F))],
        out_shape=[
            jax.ShapeDtypeStruct((N_ATOMS, F), jnp.float32),
            jax.ShapeDtypeStruct((2, F), jnp.float32),
        ],
        interpret=interpret,
    )(Y, coef1)


def _tc_pass3(A, s, coef2, interpret=False):
    return pl.pallas_call(
        _body_final,
        grid=(NBLK,),
        in_specs=[
            _row_spec(BN, F),
            _row_spec(BN, F),
            _const_spec((2, F)),
        ],
        out_specs=_row_spec(BN, F),
        out_shape=jax.ShapeDtypeStruct((N_ATOMS, F), jnp.float32),
        interpret=interpret,
    )(A, s, coef2)


def _tc_pipeline(A, G, E2, W_fc, bn1_gamma, bn1_beta, bn2_gamma, bn2_beta,
                 interpret=False):
    WsT = W_fc[:, :F].T
    WeT = W_fc[:, 2 * F:].T
    acc1, Y = _tc_pass1(A, G, E2, WsT, WeT, interpret=interpret)
    mean1 = acc1[0] / R
    var1 = acc1[1] / R - mean1 * mean1
    scale1 = bn1_gamma * lax.rsqrt(var1 + EPS)
    shift1 = bn1_beta - mean1 * scale1
    coef1 = jnp.stack([scale1, shift1])
    s, acc2 = _tc_pass2(Y, coef1, interpret=interpret)
    mean2 = acc2[0] / N_ATOMS
    var2 = acc2[1] / N_ATOMS - mean2 * mean2
    scale2 = bn2_gamma * lax.rsqrt(var2 + EPS)
    shift2 = bn2_beta - mean2 * scale2
    coef2 = jnp.stack([scale2, shift2])
    return _tc_pass3(A, s, coef2, interpret=interpret)


def kernel(atom_in_fea, nbr_fea, nbr_fea_idx, W_fc, b_fc, bn1_gamma,
           bn1_beta, bn2_gamma, bn2_beta):
    del b_fc
    idx = nbr_fea_idx.reshape(-1).astype(jnp.int32)
    idx_pad = jnp.concatenate(
        [idx, jnp.zeros((R_PAD - R,), jnp.int32)]).reshape(R_PAD // CHUNK, CHUNK)
    WnT = W_fc[:, F:2 * F].T
    P = _tc_project(atom_in_fea, WnT)
    G = _sc_gather(P, idx_pad)
    E2 = nbr_fea.reshape(R, EF)
    return _tc_pipeline(atom_in_fea, G, E2, W_fc, bn1_gamma, bn1_beta,
                        bn2_gamma, bn2_beta)


# speedup vs baseline: 1.5470x; 1.3149x over previous
import functools

import jax
import jax.numpy as jnp
from jax import lax
from jax.experimental import pallas as pl
from jax.experimental.pallas import tpu as pltpu
from jax.experimental.pallas import tpu_sc as plsc

N_ATOMS = 50000
M_NBRS = 16
F = 64
EF = 41
OUT2 = 128
EPS = 1e-5

R = N_ATOMS * M_NBRS

BC = 512
NBLK = 98
NP = NBLK * BC

NW = 32
CHUNK = 128
CHUNKS_PER_W = 200
ROWS_PER_W = CHUNKS_PER_W * CHUNK
R_PAD = NW * ROWS_PER_W
NBUF = 5


def _sc_gather(table, idx2d):
    mesh = plsc.VectorSubcoreMesh(core_axis_name="c", subcore_axis_name="s")

    @functools.partial(
        pl.kernel,
        mesh=mesh,
        out_type=jax.ShapeDtypeStruct((R_PAD, OUT2), jnp.float32),
        scratch_types=[
            pltpu.VMEM((CHUNKS_PER_W, CHUNK), jnp.int32),
        ] + [pltpu.VMEM((CHUNK, OUT2), jnp.float32)] * NBUF
          + [pltpu.SemaphoreType.DMA] * (2 * NBUF),
    )
    def k(table_hbm, idx_hbm, out_hbm, idx_v, *bufs_sems):
        bufs = bufs_sems[:NBUF]
        gsems = bufs_sems[NBUF:2 * NBUF]
        wsems = bufs_sems[2 * NBUF:]
        wid = lax.axis_index("s") * 2 + lax.axis_index("c")
        pltpu.sync_copy(idx_hbm.at[pl.ds(wid * CHUNKS_PER_W, CHUNKS_PER_W)],
                        idx_v)
        base = wid * ROWS_PER_W

        def fire(j, p):
            pltpu.async_copy(table_hbm.at[idx_v.at[j]], bufs[p], gsems[p])

        def drain_g(p):
            pltpu.make_async_copy(out_hbm.at[pl.ds(base, CHUNK)], bufs[p],
                                  gsems[p]).wait()

        def wait_w(p):
            pltpu.make_async_copy(bufs[p], out_hbm.at[pl.ds(base, CHUNK)],
                                  wsems[p]).wait()

        def write(j, p):
            pltpu.async_copy(bufs[p], out_hbm.at[pl.ds(base + j * CHUNK,
                                                       CHUNK)], wsems[p])

        for p in range(NBUF - 1):
            fire(p, p)

        nk = CHUNKS_PER_W // NBUF

        def body(k_, carry):
            for p in range(NBUF):
                j = NBUF * k_ + p
                drain_g(p)
                write(j, p)
                pn = (p + NBUF - 1) % NBUF
                if p == 0:
                    @pl.when(k_ > 0)
                    def _():
                        wait_w(pn)
                        fire(j + NBUF - 1, pn)

                    @pl.when(k_ == 0)
                    def _():
                        fire(j + NBUF - 1, pn)
                else:
                    @pl.when(k_ < nk - 1)
                    def _():
                        wait_w(pn)
                        fire(j + NBUF - 1, pn)
            return carry

        lax.fori_loop(0, nk, body, 0)
        for p in range(NBUF):
            wait_w(p)

    return k(table, idx2d)


def _softplus(x):
    return jnp.maximum(x, 0.0) + jnp.log1p(jnp.exp(-jnp.abs(x)))


def _lane_mask(i):
    lanes = lax.broadcasted_iota(jnp.int32, (1, BC), 1) + i * BC
    return lanes < N_ATOMS


def _body_pad(a_ref, t_ref):
    t_ref[...] = jnp.concatenate(
        [a_ref[...], jnp.zeros_like(a_ref[...])], axis=1)


def _body_stats(at_ref, g_ref, et_ref, ws_ref, wn_ref, we_ref,
                acc_ref, y_ref):
    i = pl.program_id(0)
    gv = g_ref[...]
    ev = et_ref[...]
    t2 = jnp.dot(ws_ref[...], at_ref[...],
                 preferred_element_type=jnp.float32)
    mask = _lane_mask(i)
    acc_s = jnp.zeros((OUT2, 1), jnp.float32)
    acc_q = jnp.zeros((OUT2, 1), jnp.float32)
    for m in range(M_NBRS):
        gm = gv[:, m, :]
        t1n = lax.dot_general(wn_ref[...], gm, (((1,), (1,)), ((), ())),
                              preferred_element_type=jnp.float32)
        em = ev[:, m, :]
        t1e = jnp.dot(we_ref[...], em,
                      preferred_element_type=jnp.float32)
        y = t2 + t1n + t1e
        y_ref[:, m, :] = y.astype(jnp.bfloat16)
        ym = jnp.where(mask, y, 0.0)
        acc_s += jnp.sum(ym, axis=1, keepdims=True)
        acc_q += jnp.sum(ym * ym, axis=1, keepdims=True)
    part = jnp.concatenate([acc_s, acc_q], axis=1)

    @pl.when(i == 0)
    def _():
        acc_ref[...] = jnp.zeros_like(acc_ref)

    acc_ref[...] += part


def _body_apply(y_ref, c1_ref, s_ref, acc2_ref):
    i = pl.program_id(0)
    c1 = c1_ref[...]
    z = (y_ref[...].astype(jnp.float32) * c1[:, 0:1][:, :, None]
         + c1[:, 1:2][:, :, None])
    f = jax.nn.sigmoid(z[:F])
    c = _softplus(z[F:])
    s = jnp.sum(f * c, axis=1)
    s_ref[...] = s
    sm = jnp.where(_lane_mask(i), s, 0.0)
    part = jnp.concatenate([jnp.sum(sm, axis=1, keepdims=True),
                            jnp.sum(sm * sm, axis=1, keepdims=True)],
                           axis=1)

    @pl.when(i == 0)
    def _():
        acc2_ref[...] = jnp.zeros_like(acc2_ref)

    acc2_ref[...] += part


def _body_final(at_ref, s_ref, c2_ref, out_ref):
    c2 = c2_ref[...]
    v = s_ref[...] * c2[:, 0:1] + c2[:, 1:2]
    out_ref[...] = _softplus(at_ref[...] + v)


def _col_spec(rows, cols):
    return pl.BlockSpec((rows, cols), lambda i: (0, i))


def _const_spec(shape):
    return pl.BlockSpec(shape, lambda i: tuple(0 for _ in shape))


PBLK = 2000


def _tc_pad(A, interpret=False):
    return pl.pallas_call(
        _body_pad,
        grid=(N_ATOMS // PBLK,),
        in_specs=[pl.BlockSpec((PBLK, F), lambda i: (i, 0))],
        out_specs=pl.BlockSpec((PBLK, OUT2), lambda i: (i, 0)),
        out_shape=jax.ShapeDtypeStruct((N_ATOMS, OUT2), jnp.float32),
        interpret=interpret,
    )(A)


def _tc_pass1(At, G3, Et, Ws, Wn, We, interpret=False):
    return pl.pallas_call(
        _body_stats,
        grid=(NBLK,),
        in_specs=[
            _col_spec(F, BC),
            pl.BlockSpec((BC, M_NBRS, OUT2), lambda i: (i, 0, 0)),
            pl.BlockSpec((EF, M_NBRS, BC), lambda i: (0, 0, i)),
            _const_spec((OUT2, F)),
            _const_spec((OUT2, OUT2)),
            _const_spec((OUT2, EF)),
        ],
        out_specs=[
            _const_spec((OUT2, 2)),
            pl.BlockSpec((OUT2, M_NBRS, BC), lambda i: (0, 0, i)),
        ],
        out_shape=[
            jax.ShapeDtypeStruct((OUT2, 2), jnp.float32),
            jax.ShapeDtypeStruct((OUT2, M_NBRS, NP), jnp.bfloat16),
        ],
        interpret=interpret,
    )(At, G3, Et, Ws, Wn, We)


def _tc_pass2(Yt, coef1, interpret=False):
    return pl.pallas_call(
        _body_apply,
        grid=(NBLK,),
        in_specs=[
            pl.BlockSpec((OUT2, M_NBRS, BC), lambda i: (0, 0, i)),
            _const_spec((OUT2, 2)),
        ],
        out_specs=[_col_spec(F, BC), _const_spec((F, 2))],
        out_shape=[
            jax.ShapeDtypeStruct((F, NP), jnp.float32),
            jax.ShapeDtypeStruct((F, 2), jnp.float32),
        ],
        interpret=interpret,
    )(Yt, coef1)


def _tc_pass3(At, St, coef2, interpret=False):
    return pl.pallas_call(
        _body_final,
        grid=(NBLK,),
        in_specs=[
            _col_spec(F, BC),
            _col_spec(F, BC),
            _const_spec((F, 2)),
        ],
        out_specs=_col_spec(F, BC),
        out_shape=jax.ShapeDtypeStruct((F, NP), jnp.float32),
        interpret=interpret,
    )(At, St, coef2)


def _tc_pipeline(At, G3, Et, W_fc, bn1_gamma, bn1_beta, bn2_gamma, bn2_beta,
                 interpret=False):
    Ws = W_fc[:, :F]
    Wn = jnp.concatenate(
        [W_fc[:, F:2 * F], jnp.zeros((OUT2, OUT2 - F), jnp.float32)], axis=1)
    We = W_fc[:, 2 * F:]
    acc1, Yt = _tc_pass1(At, G3, Et, Ws, Wn, We, interpret=interpret)
    mean1 = acc1[:, 0] / R
    var1 = acc1[:, 1] / R - mean1 * mean1
    scale1 = bn1_gamma * lax.rsqrt(var1 + EPS)
    shift1 = bn1_beta - mean1 * scale1
    coef1 = jnp.stack([scale1, shift1], axis=1)
    St, acc2 = _tc_pass2(Yt, coef1, interpret=interpret)
    mean2 = acc2[:, 0] / N_ATOMS
    var2 = acc2[:, 1] / N_ATOMS - mean2 * mean2
    scale2 = bn2_gamma * lax.rsqrt(var2 + EPS)
    shift2 = bn2_beta - mean2 * scale2
    coef2 = jnp.stack([scale2, shift2], axis=1)
    out_t = _tc_pass3(At, St, coef2, interpret=interpret)
    return jnp.transpose(out_t)[:N_ATOMS, :]


def kernel(atom_in_fea, nbr_fea, nbr_fea_idx, W_fc, b_fc, bn1_gamma,
           bn1_beta, bn2_gamma, bn2_beta):
    del b_fc
    idx = nbr_fea_idx.reshape(-1).astype(jnp.int32)
    idx_pad = jnp.concatenate(
        [idx, jnp.zeros((R_PAD - R,), jnp.int32)]).reshape(R_PAD // CHUNK, CHUNK)
    table = _tc_pad(atom_in_fea)
    G = _sc_gather(table, idx_pad)
    G3 = G.reshape(R_PAD // M_NBRS, M_NBRS, OUT2)
    At = jnp.transpose(atom_in_fea)
    Et = jnp.transpose(nbr_fea, (2, 1, 0))
    return _tc_pipeline(At, G3, Et, W_fc, bn1_gamma, bn1_beta,
                        bn2_gamma, bn2_beta)


# speedup vs baseline: 2.4692x; 1.5961x over previous
import functools

import jax
import jax.numpy as jnp
from jax import lax
from jax.experimental import pallas as pl
from jax.experimental.pallas import tpu as pltpu
from jax.experimental.pallas import tpu_sc as plsc

N_ATOMS = 50000
M_NBRS = 16
F = 64
EF = 41
OUT2 = 128
EPS = 1e-5

R = N_ATOMS * M_NBRS

BC = 512
NBLK = 98
NP = NBLK * BC

NW = 32
CHUNK = 128
HALF_BLKS = 49
NH = HALF_BLKS * BC
RH = NH * M_NBRS
ROWS_PER_W = RH // NW
SROWS = 256
NSTR = ROWS_PER_W // SROWS
NBUF = 3


def _sc_gather(table, idx2d):
    mesh = plsc.VectorSubcoreMesh(core_axis_name="c", subcore_axis_name="s")

    @functools.partial(
        pl.kernel,
        mesh=mesh,
        out_type=jax.ShapeDtypeStruct((RH, OUT2), jnp.float32),
        scratch_types=[
            pltpu.VMEM((ROWS_PER_W,), jnp.int32),
        ] + [pltpu.VMEM((SROWS, OUT2), jnp.float32)] * NBUF
          + [pltpu.SemaphoreType.DMA] * (2 * NBUF),
    )
    def k(table_hbm, idx_hbm, out_hbm, idx_v, *bufs_sems):
        bufs = bufs_sems[:NBUF]
        gsems = bufs_sems[NBUF:2 * NBUF]
        wsems = bufs_sems[2 * NBUF:]
        wid = lax.axis_index("s") * 2 + lax.axis_index("c")
        pltpu.sync_copy(idx_hbm.at[pl.ds(wid * ROWS_PER_W, ROWS_PER_W)],
                        idx_v)
        base = wid * ROWS_PER_W

        def fire(j, p):
            pltpu.async_copy(table_hbm.at[idx_v.at[pl.ds(j * SROWS, SROWS)]],
                             bufs[p], gsems[p])

        def drain_g(p):
            pltpu.make_async_copy(out_hbm.at[pl.ds(base, SROWS)], bufs[p],
                                  gsems[p]).wait()

        def wait_w(p):
            pltpu.make_async_copy(bufs[p], out_hbm.at[pl.ds(base, SROWS)],
                                  wsems[p]).wait()

        def write(j, p):
            pltpu.async_copy(bufs[p], out_hbm.at[pl.ds(base + j * SROWS,
                                                       SROWS)], wsems[p])

        fire(0, 0)
        fire(1, 1)
        nk = NSTR // NBUF

        def body(k_, carry):
            j = 3 * k_
            drain_g(0)
            write(j, 0)

            @pl.when(k_ > 0)
            def _():
                wait_w(2)
            fire(j + 2, 2)
            drain_g(1)
            write(j + 1, 1)
            wait_w(0)
            fire(j + 3, 0)
            drain_g(2)
            write(j + 2, 2)

            @pl.when(k_ < nk - 1)
            def _():
                wait_w(1)
                fire(j + 4, 1)
            return carry

        lax.fori_loop(0, nk, body, 0)
        drain_g(0)
        write(NSTR - 1, 0)
        for p in range(NBUF):
            wait_w(p)

    return k(table, idx2d)


def _softplus(x):
    return jnp.maximum(x, 0.0) + jnp.log1p(jnp.exp(-jnp.abs(x)))


def _lane_mask(i):
    lanes = lax.broadcasted_iota(jnp.int32, (1, BC), 1) + i * BC
    return lanes < N_ATOMS


def _body_pad(a_ref, t_ref):
    t_ref[...] = jnp.concatenate(
        [a_ref[...], jnp.zeros_like(a_ref[...])], axis=1)


def _body_stats(hoff, at_ref, g_ref, et_ref, ws_ref, wn_ref, we_ref,
                acc_ref, y_ref):
    i = pl.program_id(0) + hoff
    gv = g_ref[...].astype(jnp.bfloat16)
    ev = et_ref[...].astype(jnp.bfloat16)
    t2 = jnp.dot(ws_ref[...], at_ref[...].astype(jnp.bfloat16),
                 preferred_element_type=jnp.float32)
    mask = _lane_mask(i)
    acc_s = jnp.zeros((OUT2, 1), jnp.float32)
    acc_q = jnp.zeros((OUT2, 1), jnp.float32)
    for m in range(M_NBRS):
        gm = gv[:, m, :]
        t1n = lax.dot_general(wn_ref[...], gm, (((1,), (1,)), ((), ())),
                              preferred_element_type=jnp.float32)
        em = ev[:, m, :]
        t1e = jnp.dot(we_ref[...], em,
                      preferred_element_type=jnp.float32)
        y = t2 + t1n + t1e
        y_ref[:, m, :] = y.astype(jnp.bfloat16)
        ym = jnp.where(mask, y, 0.0)
        acc_s += jnp.sum(ym, axis=1, keepdims=True)
        acc_q += jnp.sum(ym * ym, axis=1, keepdims=True)
    part = jnp.concatenate([acc_s, acc_q], axis=1)

    @pl.when(pl.program_id(0) == 0)
    def _():
        acc_ref[...] = jnp.zeros_like(acc_ref)

    acc_ref[...] += part


def _body_apply(hoff, y_ref, c1_ref, s_ref, acc2_ref):
    i = pl.program_id(0) + hoff
    c1 = c1_ref[...]
    z = (y_ref[...].astype(jnp.float32) * c1[:, 0:1][:, :, None]
         + c1[:, 1:2][:, :, None])
    f = jax.nn.sigmoid(z[:F])
    c = _softplus(z[F:])
    s = jnp.sum(f * c, axis=1)
    s_ref[...] = s
    sm = jnp.where(_lane_mask(i), s, 0.0)
    part = jnp.concatenate([jnp.sum(sm, axis=1, keepdims=True),
                            jnp.sum(sm * sm, axis=1, keepdims=True)],
                           axis=1)

    @pl.when(pl.program_id(0) == 0)
    def _():
        acc2_ref[...] = jnp.zeros_like(acc2_ref)

    acc2_ref[...] += part


def _body_final(at_ref, s_ref, c2_ref, out_ref):
    c2 = c2_ref[...]
    v = s_ref[...] * c2[:, 0:1] + c2[:, 1:2]
    out_ref[...] = _softplus(at_ref[...] + v)


def _col_spec(rows, cols):
    return pl.BlockSpec((rows, cols), lambda i: (0, i))


def _const_spec(shape):
    return pl.BlockSpec(shape, lambda i: tuple(0 for _ in shape))


PBLK = 2000


def _tc_pad(A, interpret=False):
    return pl.pallas_call(
        _body_pad,
        grid=(N_ATOMS // PBLK,),
        in_specs=[pl.BlockSpec((PBLK, F), lambda i: (i, 0))],
        out_specs=pl.BlockSpec((PBLK, OUT2), lambda i: (i, 0)),
        out_shape=jax.ShapeDtypeStruct((N_ATOMS, OUT2), jnp.float32),
        interpret=interpret,
    )(A)


def _tc_pass1(half, At, G3, Et, Ws, Wn, We, interpret=False):
    hoff = half * HALF_BLKS
    return pl.pallas_call(
        functools.partial(_body_stats, hoff),
        grid=(HALF_BLKS,),
        in_specs=[
            pl.BlockSpec((F, BC), lambda i: (0, i + hoff)),
            pl.BlockSpec((BC, M_NBRS, OUT2), lambda i: (i, 0, 0)),
            pl.BlockSpec((EF, M_NBRS, BC), lambda i: (0, 0, i + hoff)),
            _const_spec((OUT2, F)),
            _const_spec((OUT2, OUT2)),
            _const_spec((OUT2, EF)),
        ],
        out_specs=[
            _const_spec((OUT2, 2)),
            pl.BlockSpec((OUT2, M_NBRS, BC), lambda i: (0, 0, i)),
        ],
        out_shape=[
            jax.ShapeDtypeStruct((OUT2, 2), jnp.float32),
            jax.ShapeDtypeStruct((OUT2, M_NBRS, NH), jnp.bfloat16),
        ],
        interpret=interpret,
    )(At, G3, Et, Ws, Wn, We)


def _tc_pass2(half, Yt, coef1, interpret=False):
    hoff = half * HALF_BLKS
    return pl.pallas_call(
        functools.partial(_body_apply, hoff),
        grid=(HALF_BLKS,),
        in_specs=[
            pl.BlockSpec((OUT2, M_NBRS, BC), lambda i: (0, 0, i)),
            _const_spec((OUT2, 2)),
        ],
        out_specs=[_col_spec(F, BC), _const_spec((F, 2))],
        out_shape=[
            jax.ShapeDtypeStruct((F, NH), jnp.float32),
            jax.ShapeDtypeStruct((F, 2), jnp.float32),
        ],
        interpret=interpret,
    )(Yt, coef1)


def _tc_pass3(half, At, St, coef2, interpret=False):
    hoff = half * HALF_BLKS
    return pl.pallas_call(
        _body_final,
        grid=(HALF_BLKS,),
        in_specs=[
            pl.BlockSpec((F, BC), lambda i: (0, i + hoff)),
            _col_spec(F, BC),
            _const_spec((F, 2)),
        ],
        out_specs=_col_spec(F, BC),
        out_shape=jax.ShapeDtypeStruct((F, NH), jnp.float32),
        interpret=interpret,
    )(At, St, coef2)


def _tc_pipeline(At, G3a, G3b, Et, W_fc, bn1_gamma, bn1_beta, bn2_gamma,
                 bn2_beta, interpret=False):
    Ws = W_fc[:, :F].astype(jnp.bfloat16)
    Wn = jnp.concatenate(
        [W_fc[:, F:2 * F], jnp.zeros((OUT2, OUT2 - F), jnp.float32)],
        axis=1).astype(jnp.bfloat16)
    We = W_fc[:, 2 * F:].astype(jnp.bfloat16)
    acc1a, Yta = _tc_pass1(0, At, G3a, Et, Ws, Wn, We, interpret=interpret)
    acc1b, Ytb = _tc_pass1(1, At, G3b, Et, Ws, Wn, We, interpret=interpret)
    acc1 = acc1a + acc1b
    mean1 = acc1[:, 0] / R
    var1 = acc1[:, 1] / R - mean1 * mean1
    scale1 = bn1_gamma * lax.rsqrt(var1 + EPS)
    shift1 = bn1_beta - mean1 * scale1
    coef1 = jnp.stack([scale1, shift1], axis=1)
    Sta, acc2a = _tc_pass2(0, Yta, coef1, interpret=interpret)
    Stb, acc2b = _tc_pass2(1, Ytb, coef1, interpret=interpret)
    acc2 = acc2a + acc2b
    mean2 = acc2[:, 0] / N_ATOMS
    var2 = acc2[:, 1] / N_ATOMS - mean2 * mean2
    scale2 = bn2_gamma * lax.rsqrt(var2 + EPS)
    shift2 = bn2_beta - mean2 * scale2
    coef2 = jnp.stack([scale2, shift2], axis=1)
    out_ta = _tc_pass3(0, At, Sta, coef2, interpret=interpret)
    out_tb = _tc_pass3(1, At, Stb, coef2, interpret=interpret)
    out = jnp.concatenate(
        [jnp.transpose(out_ta), jnp.transpose(out_tb)], axis=0)
    return out[:N_ATOMS, :]


def kernel(atom_in_fea, nbr_fea, nbr_fea_idx, W_fc, b_fc, bn1_gamma,
           bn1_beta, bn2_gamma, bn2_beta):
    del b_fc
    idx = nbr_fea_idx.reshape(-1).astype(jnp.int32)
    idx_pad = jnp.concatenate([idx, jnp.zeros((2 * RH - R,), jnp.int32)])
    table = _tc_pad(atom_in_fea)
    Ga = _sc_gather(table, idx_pad[:RH])
    Gb = _sc_gather(table, idx_pad[RH:])
    G3a = Ga.reshape(NH, M_NBRS, OUT2)
    G3b = Gb.reshape(NH, M_NBRS, OUT2)
    At = jnp.transpose(atom_in_fea)
    Et = jnp.transpose(nbr_fea, (2, 1, 0))
    return _tc_pipeline(At, G3a, G3b, Et, W_fc, bn1_gamma, bn1_beta,
                        bn2_gamma, bn2_beta)
